# baseline (speedup 1.0000x reference)
import functools

import jax
import jax.numpy as jnp
from jax import lax
from jax.experimental import pallas as pl
from jax.experimental.pallas import tpu as pltpu
from jax.experimental.pallas import tpu_sc as plsc

B = 4096
L = 200
D = 64
SCALE = 8.0

_info = plsc.get_sparse_core_info()
_NC, _NS = _info.num_cores, _info.num_subcores
NW = _NC * _NS
TOT = B * L
PER_W = TOT // NW
CHUNK = 128
NSTEPS = PER_W // CHUNK

_mesh = plsc.VectorSubcoreMesh(core_axis_name="c", subcore_axis_name="s")


@functools.partial(
    pl.kernel,
    mesh=_mesh,
    out_type=jax.ShapeDtypeStruct((TOT, D), jnp.float32),
    scratch_types=[
        pltpu.VMEM((NSTEPS, CHUNK), jnp.int32),
        pltpu.VMEM((CHUNK, D), jnp.float32),
        pltpu.VMEM((CHUNK, D), jnp.float32),
        pltpu.SemaphoreType.DMA,
    ],
    compiler_params=pltpu.CompilerParams(use_tc_tiling_on_sc=False),
)
def _emb(x_hbm, table_hbm, out_hbm, idx_v, rows_v, outb_v, sem):
    wid = lax.axis_index("s") * _NC + lax.axis_index("c")
    base = wid * PER_W
    pltpu.sync_copy(x_hbm.at[pl.ds(wid * NSTEPS, NSTEPS)], idx_v)

    def step(s, carry):
        pltpu.async_copy(table_hbm.at[idx_v.at[s]], rows_v, sem).wait()

        def scale_row(r, c2):
            for c in range(0, D, 16):
                outb_v[r, pl.ds(c, 16)] = rows_v[r, pl.ds(c, 16)] * SCALE
            return c2

        lax.fori_loop(0, CHUNK, scale_row, 0, unroll=8)
        pltpu.sync_copy(outb_v, out_hbm.at[pl.ds(base + s * CHUNK, CHUNK)])
        return carry

    lax.fori_loop(0, NSTEPS, step, 0)


def kernel(x, table):
    xf = x.reshape(TOT // CHUNK, CHUNK).astype(jnp.int32)
    out = _emb(xf, table)
    return out.reshape(B, L, D)


# speedup vs baseline: 1.3497x; 1.3497x over previous
import functools

import jax
import jax.numpy as jnp
from jax import lax
from jax.experimental import pallas as pl
from jax.experimental.pallas import tpu as pltpu
from jax.experimental.pallas import tpu_sc as plsc

B = 4096
L = 200
D = 64
SCALE = 8.0

_info = plsc.get_sparse_core_info()
_NC, _NS = _info.num_cores, _info.num_subcores
NW = _NC * _NS
TOT = B * L
PER_W = TOT // NW
CHUNK = 128
NSTEPS = PER_W // CHUNK
NBUF = 4

_mesh = plsc.VectorSubcoreMesh(core_axis_name="c", subcore_axis_name="s")


@functools.partial(
    pl.kernel,
    mesh=_mesh,
    out_type=jax.ShapeDtypeStruct((TOT, D), jnp.float32),
    scratch_types=[
        pltpu.VMEM((NSTEPS, CHUNK), jnp.int32),
        pltpu.VMEM((NBUF, CHUNK, D), jnp.float32),
        pltpu.VMEM((NBUF, CHUNK, D), jnp.float32),
    ] + [pltpu.SemaphoreType.DMA] * (2 * NBUF),
    compiler_params=pltpu.CompilerParams(use_tc_tiling_on_sc=False),
)
def _emb(x_hbm, table_hbm, out_hbm, idx_v, rows_v, outb_v, *sems):
    sem_g = sems[:NBUF]
    sem_o = sems[NBUF:]
    wid = lax.axis_index("s") * _NC + lax.axis_index("c")
    base = wid * PER_W
    pltpu.sync_copy(x_hbm.at[pl.ds(wid * NSTEPS, NSTEPS)], idx_v)

    for b in range(NBUF):
        pltpu.async_copy(table_hbm.at[idx_v.at[b]], rows_v.at[b], sem_g[b])

    def outer(i, carry):
        for b in range(NBUF):
            s = i * NBUF + b
            pltpu.make_async_copy(
                table_hbm.at[idx_v.at[s]], rows_v.at[b], sem_g[b]).wait()

            @pl.when(i > 0)
            def _wait_store():
                pltpu.make_async_copy(
                    outb_v.at[b],
                    out_hbm.at[pl.ds(base, CHUNK)],
                    sem_o[b]).wait()

            def scale_row(r, c2):
                for c in range(0, D, 16):
                    outb_v[b, r, pl.ds(c, 16)] = rows_v[b, r, pl.ds(c, 16)] * SCALE
                return c2

            lax.fori_loop(0, CHUNK, scale_row, 0, unroll=8)

            pltpu.async_copy(
                outb_v.at[b],
                out_hbm.at[pl.ds(base + s * CHUNK, CHUNK)],
                sem_o[b])

            @pl.when(s + NBUF < NSTEPS)
            def _fire_gather():
                pltpu.async_copy(
                    table_hbm.at[idx_v.at[s + NBUF]], rows_v.at[b], sem_g[b])
        return carry

    lax.fori_loop(0, NSTEPS // NBUF, outer, 0)

    for b in range(NBUF):
        pltpu.make_async_copy(
            outb_v.at[b], out_hbm.at[pl.ds(base, CHUNK)], sem_o[b]).wait()


def kernel(x, table):
    xf = x.reshape(TOT // CHUNK, CHUNK).astype(jnp.int32)
    out = _emb(xf, table)
    return out.reshape(B, L, D)


# speedup vs baseline: 1.4922x; 1.1056x over previous
import functools

import jax
import jax.numpy as jnp
from jax import lax
from jax.experimental import pallas as pl
from jax.experimental.pallas import tpu as pltpu
from jax.experimental.pallas import tpu_sc as plsc

B = 4096
L = 200
D = 64
SCALE = 8.0

_info = plsc.get_sparse_core_info()
_NC, _NS = _info.num_cores, _info.num_subcores
NW = _NC * _NS
TOT = B * L
PER_W = TOT // NW
CHUNK = 128
NSTEPS = PER_W // CHUNK
NBUF = 4

_mesh = plsc.VectorSubcoreMesh(core_axis_name="c", subcore_axis_name="s")


@functools.partial(
    pl.kernel,
    mesh=_mesh,
    out_type=jax.ShapeDtypeStruct((TOT, D), jnp.float32),
    scratch_types=[
        pltpu.VMEM((NSTEPS, CHUNK), jnp.int32),
        pltpu.VMEM((NBUF, CHUNK, D), jnp.float32),
        pltpu.VMEM((NBUF, CHUNK, D), jnp.float32),
    ] + [pltpu.SemaphoreType.DMA] * (2 * NBUF),
    compiler_params=pltpu.CompilerParams(use_tc_tiling_on_sc=False),
)
def _emb(x_hbm, table_hbm, out_hbm, idx_v, rows_v, outb_v, *sems):
    sem_g = sems[:NBUF]
    sem_o = sems[NBUF:]
    wid = lax.axis_index("s") * _NC + lax.axis_index("c")
    base = wid * PER_W
    pltpu.sync_copy(x_hbm.at[pl.ds(wid * NSTEPS, NSTEPS)], idx_v)

    for b in range(NBUF):
        pltpu.async_copy(table_hbm.at[idx_v.at[b]], rows_v.at[b], sem_g[b])

    def outer(i, carry):
        for b in range(NBUF):
            s = i * NBUF + b
            pltpu.make_async_copy(
                table_hbm.at[idx_v.at[s]], rows_v.at[b], sem_g[b]).wait()

            @pl.when(i > 0)
            def _wait_store():
                pltpu.make_async_copy(
                    outb_v.at[b],
                    out_hbm.at[pl.ds(base, CHUNK)],
                    sem_o[b]).wait()

            @plsc.parallel_loop(0, CHUNK, unroll=8)
            def _scale(r):
                for c in range(0, D, 16):
                    outb_v[b, r, pl.ds(c, 16)] = rows_v[b, r, pl.ds(c, 16)] * SCALE

            pltpu.async_copy(
                outb_v.at[b],
                out_hbm.at[pl.ds(base + s * CHUNK, CHUNK)],
                sem_o[b])

            @pl.when(s + NBUF < NSTEPS)
            def _fire_gather():
                pltpu.async_copy(
                    table_hbm.at[idx_v.at[s + NBUF]], rows_v.at[b], sem_g[b])
        return carry

    lax.fori_loop(0, NSTEPS // NBUF, outer, 0)

    for b in range(NBUF):
        pltpu.make_async_copy(
            outb_v.at[b], out_hbm.at[pl.ds(base, CHUNK)], sem_o[b]).wait()


def kernel(x, table):
    xf = x.reshape(TOT // CHUNK, CHUNK).astype(jnp.int32)
    out = _emb(xf, table)
    return out.reshape(B, L, D)
